# baseline (speedup 1.0000x reference)
import jax
import jax.numpy as jnp
from jax.experimental import pallas as pl

M = 100000
N1 = 20000
D = 128
HISTORY_LEN = 30
B = 2000
NBH = M // B
NBT = N1 // B


def _body(hf, tf, hids, hage, tids, t2r, out):
    i = pl.program_id(0)
    t2v = t2r[...]

    @pl.when(i < NBH)
    def _():
        ids = hids[...]
        found = jnp.any(ids == t2v, axis=1, keepdims=True)
        m = jnp.logical_and(
            jnp.logical_and(jnp.logical_not(found), ids != 0),
            hage[...] <= HISTORY_LEN - 1,
        )
        out[...] = hf[...] * m.astype(jnp.float32)

    @pl.when(i >= NBH)
    def _():
        ids = tids[...]
        found = jnp.any(ids == t2v, axis=1, keepdims=True)
        m = jnp.logical_and(jnp.logical_not(found), ids != 0)
        out[...] = tf[...] * m.astype(jnp.float32)


def kernel(t1_feats, hist_feats, t1_tids, t2_tids, hist_tids, hist_ages):
    hc = hist_tids[0][:, None]
    t1c = t1_tids[0][:, None]
    ages2 = hist_ages[:, None]

    grid = (NBH + NBT,)
    hist_map = lambda i: (jnp.minimum(i, NBH - 1), 0)
    t1_map = lambda i: (jnp.clip(i - NBH, 0, NBT - 1), 0)

    return pl.pallas_call(
        _body,
        grid=grid,
        in_specs=[
            pl.BlockSpec((B, D), hist_map),
            pl.BlockSpec((B, D), t1_map),
            pl.BlockSpec((B, 1), hist_map),
            pl.BlockSpec((B, 1), hist_map),
            pl.BlockSpec((B, 1), t1_map),
            pl.BlockSpec((1, 512), lambda i: (0, 0)),
        ],
        out_specs=pl.BlockSpec((B, D), lambda i: (i, 0)),
        out_shape=jax.ShapeDtypeStruct((M + N1, D), jnp.float32),
    )(hist_feats, t1_feats, hc, ages2, t1c, t2_tids)


# speedup vs baseline: 1.1034x; 1.1034x over previous
import jax
import jax.numpy as jnp
from jax import lax
from jax.experimental import pallas as pl
from jax.experimental.pallas import tpu as pltpu
from jax.experimental.pallas import tpu_sc as plsc

M = 100000
N1 = 20000
D = 128
HISTORY_LEN = 30
TID_RANGE = 30000

NW = 32
HP = 100352
TP = 20480
CH = HP // NW
CT = TP // NW
NSET = 512

B = 2000
NBH = M // B
NBT = N1 // B


def _sc_mask_body(t2_hbm, htids_hbm, hages_hbm, ttids_hbm,
                  mask_h_hbm, mask_t_hbm,
                  table_v, t2_v, tids_v, ages_v, mask_v, ttids_v, maskt_v):
    wid = lax.axis_index("s") * 2 + lax.axis_index("c")

    def _zero(i, _):
        table_v[pl.ds(i * 16, 16)] = jnp.zeros((16,), jnp.int32)
        return 0
    lax.fori_loop(0, TID_RANGE // 16, _zero, 0)

    pltpu.sync_copy(t2_hbm, t2_v)

    def _scatter(j, _):
        idx = t2_v[pl.ds(j * 16, 16)]
        plsc.store_scatter(table_v, [idx], jnp.ones((16,), jnp.int32))
        return 0
    lax.fori_loop(0, NSET // 16, _scatter, 0)

    base_h = wid * CH
    pltpu.sync_copy(htids_hbm.at[pl.ds(base_h, CH)], tids_v)
    pltpu.sync_copy(hages_hbm.at[pl.ds(base_h, CH)], ages_v)

    def _hist(k, _):
        sl = pl.ds(k * 16, 16)
        tid = tids_v[sl]
        hit = plsc.load_gather(table_v, [tid])
        age = ages_v[sl]
        keep = jnp.logical_and(
            jnp.logical_and(tid != 0, hit == 0), age <= HISTORY_LEN - 1)
        mask_v[sl] = jnp.where(keep, 1.0, 0.0).astype(jnp.float32)
        return 0
    lax.fori_loop(0, CH // 16, _hist, 0)
    pltpu.sync_copy(mask_v, mask_h_hbm.at[pl.ds(base_h, CH)])

    base_t = wid * CT
    pltpu.sync_copy(ttids_hbm.at[pl.ds(base_t, CT)], ttids_v)

    def _det(k, _):
        sl = pl.ds(k * 16, 16)
        tid = ttids_v[sl]
        hit = plsc.load_gather(table_v, [tid])
        keep = jnp.logical_and(tid != 0, hit == 0)
        maskt_v[sl] = jnp.where(keep, 1.0, 0.0).astype(jnp.float32)
        return 0
    lax.fori_loop(0, CT // 16, _det, 0)
    pltpu.sync_copy(maskt_v, mask_t_hbm.at[pl.ds(base_t, CT)])


_sc_masks = pl.kernel(
    _sc_mask_body,
    out_type=(
        jax.ShapeDtypeStruct((HP,), jnp.float32),
        jax.ShapeDtypeStruct((TP,), jnp.float32),
    ),
    mesh=plsc.VectorSubcoreMesh(core_axis_name="c", subcore_axis_name="s"),
    compiler_params=pltpu.CompilerParams(needs_layout_passes=False),
    scratch_types=[
        pltpu.VMEM((TID_RANGE,), jnp.int32),
        pltpu.VMEM((NSET,), jnp.int32),
        pltpu.VMEM((CH,), jnp.int32),
        pltpu.VMEM((CH,), jnp.int32),
        pltpu.VMEM((CH,), jnp.float32),
        pltpu.VMEM((CT,), jnp.int32),
        pltpu.VMEM((CT,), jnp.float32),
    ],
)


def _tc_body(hf, tf, mh, mt, out):
    i = pl.program_id(0)

    @pl.when(i < NBH)
    def _():
        out[...] = hf[...] * mh[...]

    @pl.when(i >= NBH)
    def _():
        out[...] = tf[...] * mt[...]


def kernel(t1_feats, hist_feats, t1_tids, t2_tids, hist_tids, hist_ages):
    t2 = t2_tids[0]
    htids = jnp.pad(hist_tids[0], (0, HP - M))
    hages = jnp.pad(hist_ages, (0, HP - M))
    ttids = jnp.pad(t1_tids[0], (0, TP - N1))

    mask_h_p, mask_t_p = _sc_masks(t2, htids, hages, ttids)
    mh2 = mask_h_p[:M, None]
    mt2 = mask_t_p[:N1, None]

    hist_map = lambda i: (jnp.minimum(i, NBH - 1), 0)
    t1_map = lambda i: (jnp.clip(i - NBH, 0, NBT - 1), 0)

    return pl.pallas_call(
        _tc_body,
        grid=(NBH + NBT,),
        in_specs=[
            pl.BlockSpec((B, D), hist_map),
            pl.BlockSpec((B, D), t1_map),
            pl.BlockSpec((B, 1), hist_map),
            pl.BlockSpec((B, 1), t1_map),
        ],
        out_specs=pl.BlockSpec((B, D), lambda i: (i, 0)),
        out_shape=jax.ShapeDtypeStruct((M + N1, D), jnp.float32),
    )(hist_feats, t1_feats, mh2, mt2)


# speedup vs baseline: 1.3949x; 1.2643x over previous
import jax
import jax.numpy as jnp
from jax import lax
from jax.experimental import pallas as pl
from jax.experimental.pallas import tpu as pltpu
from jax.experimental.pallas import tpu_sc as plsc

M = 100000
N1 = 20000
D = 128
HISTORY_LEN = 30
TID_RANGE = 30000

NW = 32
HP = 100352
TP = 20480
CH = HP // NW
CT = TP // NW
NSET = 512

B = 4000
NBH = M // B
NBT = N1 // B


def _sc_mask_body(zeros_hbm, t2_hbm, htids_hbm, hages_hbm, ttids_hbm,
                  mask_h_hbm, mask_t_hbm,
                  table_v, t2_v, tids_v, ages_v, mask_v, ttids_v, maskt_v):
    wid = lax.axis_index("s") * 2 + lax.axis_index("c")

    pltpu.sync_copy(zeros_hbm, table_v)

    pltpu.sync_copy(t2_hbm, t2_v)

    def _scatter(j, _):
        idx = t2_v[pl.ds(j * 16, 16)]
        plsc.store_scatter(table_v, [idx], jnp.ones((16,), jnp.int32))
        return 0
    lax.fori_loop(0, NSET // 16, _scatter, 0)

    base_h = wid * CH
    pltpu.sync_copy(htids_hbm.at[pl.ds(base_h, CH)], tids_v)
    pltpu.sync_copy(hages_hbm.at[pl.ds(base_h, CH)], ages_v)

    def _hist(k, _):
        sl = pl.ds(k * 16, 16)
        tid = tids_v[sl]
        hit = plsc.load_gather(table_v, [tid])
        age = ages_v[sl]
        keep = jnp.logical_and(
            jnp.logical_and(tid != 0, hit == 0), age <= HISTORY_LEN - 1)
        mask_v[sl] = jnp.where(keep, 1.0, 0.0).astype(jnp.float32)
        return 0
    lax.fori_loop(0, CH // 16, _hist, 0)
    pltpu.sync_copy(mask_v, mask_h_hbm.at[pl.ds(base_h, CH)])

    base_t = wid * CT
    pltpu.sync_copy(ttids_hbm.at[pl.ds(base_t, CT)], ttids_v)

    def _det(k, _):
        sl = pl.ds(k * 16, 16)
        tid = ttids_v[sl]
        hit = plsc.load_gather(table_v, [tid])
        keep = jnp.logical_and(tid != 0, hit == 0)
        maskt_v[sl] = jnp.where(keep, 1.0, 0.0).astype(jnp.float32)
        return 0
    lax.fori_loop(0, CT // 16, _det, 0)
    pltpu.sync_copy(maskt_v, mask_t_hbm.at[pl.ds(base_t, CT)])


_sc_masks = pl.kernel(
    _sc_mask_body,
    out_type=(
        jax.ShapeDtypeStruct((HP,), jnp.float32),
        jax.ShapeDtypeStruct((TP,), jnp.float32),
    ),
    mesh=plsc.VectorSubcoreMesh(core_axis_name="c", subcore_axis_name="s"),
    compiler_params=pltpu.CompilerParams(needs_layout_passes=False),
    scratch_types=[
        pltpu.VMEM((TID_RANGE,), jnp.int32),
        pltpu.VMEM((NSET,), jnp.int32),
        pltpu.VMEM((CH,), jnp.int32),
        pltpu.VMEM((CH,), jnp.int32),
        pltpu.VMEM((CH,), jnp.float32),
        pltpu.VMEM((CT,), jnp.int32),
        pltpu.VMEM((CT,), jnp.float32),
    ],
)


def _tc_body(hf, tf, mh, mt, out):
    i = pl.program_id(0)

    @pl.when(i < NBH)
    def _():
        out[...] = hf[...] * mh[...]

    @pl.when(i >= NBH)
    def _():
        out[...] = tf[...] * mt[...]


def kernel(t1_feats, hist_feats, t1_tids, t2_tids, hist_tids, hist_ages):
    t2 = t2_tids[0]
    htids = jnp.pad(hist_tids[0], (0, HP - M))
    hages = jnp.pad(hist_ages, (0, HP - M))
    ttids = jnp.pad(t1_tids[0], (0, TP - N1))
    ztable = jnp.zeros((TID_RANGE,), jnp.int32)

    mask_h_p, mask_t_p = _sc_masks(ztable, t2, htids, hages, ttids)
    mh2 = mask_h_p.reshape(HP, 1)
    mt2 = mask_t_p.reshape(TP, 1)

    hist_map = lambda i: (jnp.minimum(i, NBH - 1), 0)
    t1_map = lambda i: (jnp.clip(i - NBH, 0, NBT - 1), 0)

    return pl.pallas_call(
        _tc_body,
        grid=(NBH + NBT,),
        in_specs=[
            pl.BlockSpec((B, D), hist_map),
            pl.BlockSpec((B, D), t1_map),
            pl.BlockSpec((B, 1), hist_map),
            pl.BlockSpec((B, 1), t1_map),
        ],
        out_specs=pl.BlockSpec((B, D), lambda i: (i, 0)),
        out_shape=jax.ShapeDtypeStruct((M + N1, D), jnp.float32),
    )(hist_feats, t1_feats, mh2, mt2)


# speedup vs baseline: 1.4092x; 1.0102x over previous
import jax
import jax.numpy as jnp
from jax import lax
from jax.experimental import pallas as pl
from jax.experimental.pallas import tpu as pltpu
from jax.experimental.pallas import tpu_sc as plsc

M = 100000
N1 = 20000
D = 128
HISTORY_LEN = 30
TID_RANGE = 30000

NW = 32
HP = 100352
TP = 20480
CH = HP // NW
CT = TP // NW
NSET = 512

B = 5000
NBH = M // B
NBT = N1 // B


def _sc_mask_body(zeros_hbm, t2_hbm, htids_hbm, hages_hbm, ttids_hbm,
                  mask_h_hbm, mask_t_hbm,
                  table_v, t2_v, tids_v, ages_v, mask_v, ttids_v, maskt_v):
    wid = lax.axis_index("s") * 2 + lax.axis_index("c")

    pltpu.sync_copy(zeros_hbm, table_v)

    pltpu.sync_copy(t2_hbm, t2_v)

    def _scatter(j, _):
        idx = t2_v[pl.ds(j * 16, 16)]
        plsc.store_scatter(table_v, [idx], jnp.ones((16,), jnp.int32))
        return 0
    lax.fori_loop(0, NSET // 16, _scatter, 0)

    base_h = wid * CH
    pltpu.sync_copy(htids_hbm.at[pl.ds(base_h, CH)], tids_v)
    pltpu.sync_copy(hages_hbm.at[pl.ds(base_h, CH)], ages_v)

    def _hist(k, _):
        sl = pl.ds(k * 16, 16)
        tid = tids_v[sl]
        hit = plsc.load_gather(table_v, [tid])
        age = ages_v[sl]
        keep = jnp.logical_and(
            jnp.logical_and(tid != 0, hit == 0), age <= HISTORY_LEN - 1)
        mask_v[sl] = jnp.where(keep, 1.0, 0.0).astype(jnp.float32)
        return 0
    lax.fori_loop(0, CH // 16, _hist, 0)
    pltpu.sync_copy(mask_v, mask_h_hbm.at[pl.ds(base_h, CH)])

    base_t = wid * CT
    pltpu.sync_copy(ttids_hbm.at[pl.ds(base_t, CT)], ttids_v)

    def _det(k, _):
        sl = pl.ds(k * 16, 16)
        tid = ttids_v[sl]
        hit = plsc.load_gather(table_v, [tid])
        keep = jnp.logical_and(tid != 0, hit == 0)
        maskt_v[sl] = jnp.where(keep, 1.0, 0.0).astype(jnp.float32)
        return 0
    lax.fori_loop(0, CT // 16, _det, 0)
    pltpu.sync_copy(maskt_v, mask_t_hbm.at[pl.ds(base_t, CT)])


_sc_masks = pl.kernel(
    _sc_mask_body,
    out_type=(
        jax.ShapeDtypeStruct((HP,), jnp.float32),
        jax.ShapeDtypeStruct((TP,), jnp.float32),
    ),
    mesh=plsc.VectorSubcoreMesh(core_axis_name="c", subcore_axis_name="s"),
    compiler_params=pltpu.CompilerParams(needs_layout_passes=False),
    scratch_types=[
        pltpu.VMEM((TID_RANGE,), jnp.int32),
        pltpu.VMEM((NSET,), jnp.int32),
        pltpu.VMEM((CH,), jnp.int32),
        pltpu.VMEM((CH,), jnp.int32),
        pltpu.VMEM((CH,), jnp.float32),
        pltpu.VMEM((CT,), jnp.int32),
        pltpu.VMEM((CT,), jnp.float32),
    ],
)


def _tc_body(hf, tf, mh, mt, out):
    i = pl.program_id(0)

    @pl.when(i < NBH)
    def _():
        out[...] = hf[...] * mh[...]

    @pl.when(i >= NBH)
    def _():
        out[...] = tf[...] * mt[...]


def kernel(t1_feats, hist_feats, t1_tids, t2_tids, hist_tids, hist_ages):
    t2 = t2_tids[0]
    htids = jnp.pad(hist_tids[0], (0, HP - M))
    hages = jnp.pad(hist_ages, (0, HP - M))
    ttids = jnp.pad(t1_tids[0], (0, TP - N1))
    ztable = jnp.zeros((TID_RANGE,), jnp.int32)

    mask_h_p, mask_t_p = _sc_masks(ztable, t2, htids, hages, ttids)
    mh2 = mask_h_p.reshape(HP, 1)
    mt2 = mask_t_p.reshape(TP, 1)

    hist_map = lambda i: (jnp.minimum(i, NBH - 1), 0)
    t1_map = lambda i: (jnp.clip(i - NBH, 0, NBT - 1), 0)

    return pl.pallas_call(
        _tc_body,
        grid=(NBH + NBT,),
        in_specs=[
            pl.BlockSpec((B, D), hist_map),
            pl.BlockSpec((B, D), t1_map),
            pl.BlockSpec((B, 1), hist_map),
            pl.BlockSpec((B, 1), t1_map),
        ],
        out_specs=pl.BlockSpec((B, D), lambda i: (i, 0)),
        out_shape=jax.ShapeDtypeStruct((M + N1, D), jnp.float32),
    )(hist_feats, t1_feats, mh2, mt2)


# speedup vs baseline: 2.0950x; 1.4867x over previous
import jax
import jax.numpy as jnp
from jax import lax
from jax.experimental import pallas as pl
from jax.experimental.pallas import tpu as pltpu
from jax.experimental.pallas import tpu_sc as plsc

M = 100000
N1 = 20000
D = 128
HISTORY_LEN = 30
TID_RANGE = 30000

NW = 32
HP = 100352
TP = 20480
CH = HP // NW
CT = TP // NW
NSET = 512
NDROP = CH + CT + 16

SH = 112
NSEG_H = CH // SH
ST = 128
NSEG_T = CT // ST
NBUF = 4


def _sc_body(zt_hbm, zr_hbm, t2_hbm, htids_hbm, hages_hbm, ttids_hbm,
             hfeat_hbm, tfeat_hbm, out_hbm,
             table_v, t2_v, tids_v, ages_v, drop_v, zrows_v,
             b0, b1, b2, b3, i0, i1, i2, i3, o0, o1, o2, o3, ssem):
    wid = lax.axis_index("s") * 2 + lax.axis_index("c")
    bufs = [b0, b1, b2, b3]
    ins = [i0, i1, i2, i3]
    outs = [o0, o1, o2, o3]

    pltpu.sync_copy(zt_hbm, table_v)
    pltpu.sync_copy(zr_hbm, zrows_v)
    pltpu.sync_copy(t2_hbm, t2_v)

    def _scatter(j, _):
        idx = t2_v[pl.ds(j * 16, 16)]
        plsc.store_scatter(table_v, [idx], jnp.ones((16,), jnp.float32))
        return 0
    lax.fori_loop(0, NSET // 16, _scatter, 0)

    base_h = wid * CH
    pltpu.sync_copy(htids_hbm.at[pl.ds(base_h, CH)], tids_v)
    pltpu.sync_copy(hages_hbm.at[pl.ds(base_h, CH)], ages_v)
    lanes = lax.iota(jnp.int32, 16)

    def _hist(k, cnt):
        sl = pl.ds(k * 16, 16)
        tid = tids_v[sl]
        hit = plsc.load_gather(table_v, [tid])
        age = ages_v[sl]
        keep = jnp.logical_and(
            jnp.logical_and(tid != 0, hit == 0.0), age <= HISTORY_LEN - 1)
        rows = base_h + k * 16 + lanes
        dropm = jnp.logical_and(jnp.logical_not(keep), rows < M)
        plsc.store_compressed(drop_v.at[pl.ds(cnt, 16)], rows, mask=dropm)
        return cnt + jnp.sum(dropm.astype(jnp.int32))
    cnt = lax.fori_loop(0, CH // 16, _hist, jnp.int32(0))

    base_t = wid * CT
    pltpu.sync_copy(ttids_hbm.at[pl.ds(base_t, CT)], tids_v.at[pl.ds(0, CT)])

    def _det(k, cnt):
        sl = pl.ds(k * 16, 16)
        tid = tids_v[sl]
        hit = plsc.load_gather(table_v, [tid])
        keep = jnp.logical_and(tid != 0, hit == 0.0)
        rows = base_t + k * 16 + lanes
        dropm = jnp.logical_and(jnp.logical_not(keep), rows < N1)
        plsc.store_compressed(drop_v.at[pl.ds(cnt, 16)], rows + M, mask=dropm)
        return cnt + jnp.sum(dropm.astype(jnp.int32))
    cnt = lax.fori_loop(0, CT // 16, _det, cnt)

    def h_src(g):
        start = jnp.minimum(base_h + g * SH, M - SH)
        return hfeat_hbm.at[pl.ds(start, SH), :]

    def h_dst(g):
        start = jnp.minimum(base_h + g * SH, M - SH)
        return out_hbm.at[pl.ds(start, SH), :]

    for b in range(NBUF):
        pltpu.async_copy(h_src(b), bufs[b].at[pl.ds(0, SH), :], ins[b])

    @pl.loop(0, NSEG_H // NBUF)
    def _copy_h(o):
        for b in range(NBUF):
            g = o * NBUF + b
            pltpu.make_async_copy(h_src(0), bufs[b].at[pl.ds(0, SH), :],
                                  ins[b]).wait()
            pltpu.async_copy(bufs[b].at[pl.ds(0, SH), :], h_dst(g), outs[b])

            @pl.when(g + NBUF < NSEG_H)
            def _():
                pltpu.make_async_copy(bufs[b].at[pl.ds(0, SH), :], h_dst(0),
                                      outs[b]).wait()
                pltpu.async_copy(h_src(g + NBUF), bufs[b].at[pl.ds(0, SH), :],
                                 ins[b])

    for b in range(NBUF):
        pltpu.make_async_copy(bufs[b].at[pl.ds(0, SH), :], h_dst(0),
                              outs[b]).wait()

    def t_src(g):
        start = jnp.minimum(base_t + g * ST, N1 - ST)
        return tfeat_hbm.at[pl.ds(start, ST), :]

    def t_dst(g):
        start = jnp.minimum(base_t + g * ST, N1 - ST)
        return out_hbm.at[pl.ds(M + start, ST), :]

    for b in range(NBUF):
        pltpu.async_copy(t_src(b), bufs[b], ins[b])
    for g in range(NSEG_T):
        b = g % NBUF
        pltpu.make_async_copy(t_src(0), bufs[b], ins[b]).wait()
        pltpu.async_copy(bufs[b], t_dst(g), outs[b])
        if g + NBUF < NSEG_T:
            pltpu.make_async_copy(bufs[b], t_dst(0), outs[b]).wait()
            pltpu.async_copy(t_src(g + NBUF), bufs[b], ins[b])
    for g in range(max(NSEG_T - NBUF, 0), NSEG_T):
        pltpu.make_async_copy(bufs[g % NBUF], t_dst(0), outs[g % NBUF]).wait()

    nfull = cnt // 16
    rem = cnt - nfull * 16

    def _fire(c, _):
        idx = drop_v[pl.ds(c * 16, 16)]
        pltpu.async_copy(zrows_v, out_hbm.at[idx], ssem)
        return 0
    lax.fori_loop(0, nfull, _fire, 0)

    @pl.when(rem > 0)
    def _():
        head = plsc.load_gather(drop_v, [jnp.full((16,), nfull * 16, jnp.int32)])
        tail = drop_v[pl.ds(nfull * 16, 16)]
        idx = jnp.where(lanes < rem, tail, head)
        pltpu.async_copy(zrows_v, out_hbm.at[idx], ssem)

    nchunks = nfull + jnp.where(rem > 0, 1, 0).astype(jnp.int32)

    def _drain(c, _):
        pltpu.make_async_copy(zrows_v, out_hbm.at[jnp.zeros((16,), jnp.int32)],
                              ssem).wait()
        return 0
    lax.fori_loop(0, nchunks, _drain, 0)


_sc_full = pl.kernel(
    _sc_body,
    out_type=jax.ShapeDtypeStruct((M + N1, D), jnp.float32),
    mesh=plsc.VectorSubcoreMesh(core_axis_name="c", subcore_axis_name="s"),
    compiler_params=pltpu.CompilerParams(needs_layout_passes=False),
    scratch_types=[
        pltpu.VMEM((TID_RANGE,), jnp.float32),
        pltpu.VMEM((NSET,), jnp.int32),
        pltpu.VMEM((CH,), jnp.int32),
        pltpu.VMEM((CH,), jnp.int32),
        pltpu.VMEM((NDROP,), jnp.int32),
        pltpu.VMEM((16, D), jnp.float32),
    ]
    + [pltpu.VMEM((ST, D), jnp.float32)] * NBUF
    + [pltpu.SemaphoreType.DMA] * (2 * NBUF + 1),
)


def kernel(t1_feats, hist_feats, t1_tids, t2_tids, hist_tids, hist_ages):
    t2 = t2_tids[0]
    htids = jnp.pad(hist_tids[0], (0, HP - M))
    hages = jnp.pad(hist_ages, (0, HP - M))
    ttids = jnp.pad(t1_tids[0], (0, TP - N1))
    ztable = jnp.zeros((TID_RANGE,), jnp.float32)
    zrows = jnp.zeros((16, D), jnp.float32)
    return _sc_full(ztable, zrows, t2, htids, hages, ttids,
                    hist_feats, t1_feats)


# speedup vs baseline: 2.1522x; 1.0273x over previous
import jax
import jax.numpy as jnp
from jax import lax
from jax.experimental import pallas as pl
from jax.experimental.pallas import tpu as pltpu
from jax.experimental.pallas import tpu_sc as plsc

M = 100000
N1 = 20000
D = 128
HISTORY_LEN = 30
TID_RANGE = 30000

NW = 32
CH = 3136
CT = 640
NSET = 512
NDROP = CH + CT + 16

SH = 112
NSEG_H = CH // SH
ST = 128
NSEG_T = CT // ST
NBUF = 4


def _sc_body(zt_hbm, zr_hbm, t2_hbm, htids_hbm, hages_hbm, ttids_hbm,
             hfeat_hbm, tfeat_hbm, out_hbm,
             table_v, t2_v, tids_v, ages_v, drop_v, zrows_v,
             b0, b1, b2, b3, i0, i1, i2, i3, o0, o1, o2, o3, ssem):
    wid = lax.axis_index("s") * 2 + lax.axis_index("c")
    bufs = [b0, b1, b2, b3]
    ins = [i0, i1, i2, i3]
    outs = [o0, o1, o2, o3]

    pltpu.sync_copy(zt_hbm, table_v)
    pltpu.sync_copy(zr_hbm, zrows_v)
    pltpu.sync_copy(t2_hbm, t2_v)

    def _scatter(j, _):
        idx = t2_v[pl.ds(j * 16, 16)]
        plsc.store_scatter(table_v, [idx], jnp.ones((16,), jnp.float32))
        return 0
    lax.fori_loop(0, NSET // 16, _scatter, 0)

    base_h = wid * CH
    start_h = jnp.minimum(base_h, M - CH)
    pltpu.sync_copy(htids_hbm.at[pl.ds(start_h, CH)], tids_v)
    pltpu.sync_copy(hages_hbm.at[pl.ds(start_h, CH)], ages_v)
    lanes = lax.iota(jnp.int32, 16)

    def _hist(k, cnt):
        sl = pl.ds(k * 16, 16)
        tid = tids_v[sl]
        hit = plsc.load_gather(table_v, [tid])
        age = ages_v[sl]
        keep = jnp.logical_and(
            jnp.logical_and(tid != 0, hit == 0.0), age <= HISTORY_LEN - 1)
        rows = start_h + k * 16 + lanes
        dropm = jnp.logical_not(keep)
        plsc.store_compressed(drop_v.at[pl.ds(cnt, 16)], rows, mask=dropm)
        return cnt + jnp.sum(dropm.astype(jnp.int32))
    cnt = lax.fori_loop(0, CH // 16, _hist, jnp.int32(0))

    base_t = wid * CT
    start_t = jnp.minimum(base_t, N1 - CT)
    pltpu.sync_copy(ttids_hbm.at[pl.ds(start_t, CT)], tids_v.at[pl.ds(0, CT)])

    def _det(k, cnt):
        sl = pl.ds(k * 16, 16)
        tid = tids_v[sl]
        hit = plsc.load_gather(table_v, [tid])
        keep = jnp.logical_and(tid != 0, hit == 0.0)
        rows = start_t + k * 16 + lanes
        dropm = jnp.logical_not(keep)
        plsc.store_compressed(drop_v.at[pl.ds(cnt, 16)], rows + M, mask=dropm)
        return cnt + jnp.sum(dropm.astype(jnp.int32))
    cnt = lax.fori_loop(0, CT // 16, _det, cnt)

    def h_src(g):
        start = jnp.minimum(base_h + g * SH, M - SH)
        return hfeat_hbm.at[pl.ds(start, SH), :]

    def h_dst(g):
        start = jnp.minimum(base_h + g * SH, M - SH)
        return out_hbm.at[pl.ds(start, SH), :]

    for b in range(NBUF):
        pltpu.async_copy(h_src(b), bufs[b].at[pl.ds(0, SH), :], ins[b])

    @pl.loop(0, NSEG_H // NBUF)
    def _copy_h(o):
        for b in range(NBUF):
            g = o * NBUF + b
            pltpu.make_async_copy(h_src(0), bufs[b].at[pl.ds(0, SH), :],
                                  ins[b]).wait()
            pltpu.async_copy(bufs[b].at[pl.ds(0, SH), :], h_dst(g), outs[b])

            @pl.when(g + NBUF < NSEG_H)
            def _():
                pltpu.make_async_copy(bufs[b].at[pl.ds(0, SH), :], h_dst(0),
                                      outs[b]).wait()
                pltpu.async_copy(h_src(g + NBUF), bufs[b].at[pl.ds(0, SH), :],
                                 ins[b])

    for b in range(NBUF):
        pltpu.make_async_copy(bufs[b].at[pl.ds(0, SH), :], h_dst(0),
                              outs[b]).wait()

    def t_src(g):
        start = jnp.minimum(base_t + g * ST, N1 - ST)
        return tfeat_hbm.at[pl.ds(start, ST), :]

    def t_dst(g):
        start = jnp.minimum(base_t + g * ST, N1 - ST)
        return out_hbm.at[pl.ds(M + start, ST), :]

    for b in range(NBUF):
        pltpu.async_copy(t_src(b), bufs[b], ins[b])
    for g in range(NSEG_T):
        b = g % NBUF
        pltpu.make_async_copy(t_src(0), bufs[b], ins[b]).wait()
        pltpu.async_copy(bufs[b], t_dst(g), outs[b])
        if g + NBUF < NSEG_T:
            pltpu.make_async_copy(bufs[b], t_dst(0), outs[b]).wait()
            pltpu.async_copy(t_src(g + NBUF), bufs[b], ins[b])
    for g in range(max(NSEG_T - NBUF, 0), NSEG_T):
        pltpu.make_async_copy(bufs[g % NBUF], t_dst(0), outs[g % NBUF]).wait()

    nfull = cnt // 16
    rem = cnt - nfull * 16

    def _fire(c, _):
        idx = drop_v[pl.ds(c * 16, 16)]
        pltpu.async_copy(zrows_v, out_hbm.at[idx], ssem)
        return 0
    lax.fori_loop(0, nfull, _fire, 0)

    @pl.when(rem > 0)
    def _():
        head = plsc.load_gather(drop_v, [jnp.full((16,), nfull * 16, jnp.int32)])
        tail = drop_v[pl.ds(nfull * 16, 16)]
        idx = jnp.where(lanes < rem, tail, head)
        pltpu.async_copy(zrows_v, out_hbm.at[idx], ssem)

    nchunks = nfull + jnp.where(rem > 0, 1, 0).astype(jnp.int32)

    def _drain(c, _):
        pltpu.make_async_copy(zrows_v, out_hbm.at[jnp.zeros((16,), jnp.int32)],
                              ssem).wait()
        return 0
    lax.fori_loop(0, nchunks, _drain, 0)


_sc_full = pl.kernel(
    _sc_body,
    out_type=jax.ShapeDtypeStruct((M + N1, D), jnp.float32),
    mesh=plsc.VectorSubcoreMesh(core_axis_name="c", subcore_axis_name="s"),
    compiler_params=pltpu.CompilerParams(needs_layout_passes=False),
    scratch_types=[
        pltpu.VMEM((TID_RANGE,), jnp.float32),
        pltpu.VMEM((NSET,), jnp.int32),
        pltpu.VMEM((CH,), jnp.int32),
        pltpu.VMEM((CH,), jnp.int32),
        pltpu.VMEM((NDROP,), jnp.int32),
        pltpu.VMEM((16, D), jnp.float32),
    ]
    + [pltpu.VMEM((ST, D), jnp.float32)] * NBUF
    + [pltpu.SemaphoreType.DMA] * (2 * NBUF + 1),
)


def kernel(t1_feats, hist_feats, t1_tids, t2_tids, hist_tids, hist_ages):
    ztable = jnp.zeros((TID_RANGE,), jnp.float32)
    zrows = jnp.zeros((16, D), jnp.float32)
    return _sc_full(ztable, zrows, t2_tids[0], hist_tids[0], hist_ages,
                    t1_tids[0], hist_feats, t1_feats)


# speedup vs baseline: 2.2488x; 1.0449x over previous
import jax
import jax.numpy as jnp
from jax import lax
from jax.experimental import pallas as pl
from jax.experimental.pallas import tpu as pltpu
from jax.experimental.pallas import tpu_sc as plsc

M = 100000
N1 = 20000
D = 128
HISTORY_LEN = 30
TID_RANGE = 30000

NW = 32
CH = 3136
CT = 640
NSET = 512
NDROP = CH + CT + 16

SH = 112
NSEG_H = CH // SH
ST = 128
NSEG_T = CT // ST
NBUF = 4


def _sc_body(zt_hbm, zr_hbm, t2_hbm, htids_hbm, hages_hbm, ttids_hbm,
             hfeat_hbm, tfeat_hbm, out_hbm,
             table_v, t2_v, tids_v, ages_v, ttids_v, drop_v, zrows_v,
             b0, b1, b2, b3, i0, i1, i2, i3, o0, o1, o2, o3, ssem,
             m0, m1, m2, m3, m4, m5):
    wid = lax.axis_index("s") * 2 + lax.axis_index("c")
    bufs = [b0, b1, b2, b3]
    ins = [i0, i1, i2, i3]
    outs = [o0, o1, o2, o3]
    base_h = wid * CH
    start_h = jnp.minimum(base_h, M - CH)
    base_t = wid * CT
    start_t = jnp.minimum(base_t, N1 - CT)

    pltpu.async_copy(zt_hbm, table_v, m0)
    pltpu.async_copy(t2_hbm, t2_v, m1)
    pltpu.async_copy(htids_hbm.at[pl.ds(start_h, CH)], tids_v, m2)
    pltpu.async_copy(hages_hbm.at[pl.ds(start_h, CH)], ages_v, m3)
    pltpu.async_copy(ttids_hbm.at[pl.ds(start_t, CT)], ttids_v, m4)
    pltpu.async_copy(zr_hbm, zrows_v, m5)

    def h_src(g):
        start = jnp.minimum(base_h + g * SH, M - SH)
        return hfeat_hbm.at[pl.ds(start, SH), :]

    def h_dst(g):
        start = jnp.minimum(base_h + g * SH, M - SH)
        return out_hbm.at[pl.ds(start, SH), :]

    for b in range(NBUF):
        pltpu.async_copy(h_src(b), bufs[b].at[pl.ds(0, SH), :], ins[b])

    pltpu.make_async_copy(zt_hbm, table_v, m0).wait()
    pltpu.make_async_copy(t2_hbm, t2_v, m1).wait()

    def _scatter(j, _):
        idx = t2_v[pl.ds(j * 16, 16)]
        plsc.store_scatter(table_v, [idx], jnp.ones((16,), jnp.float32))
        return 0
    lax.fori_loop(0, NSET // 16, _scatter, 0)

    pltpu.make_async_copy(htids_hbm.at[pl.ds(start_h, CH)], tids_v, m2).wait()
    pltpu.make_async_copy(hages_hbm.at[pl.ds(start_h, CH)], ages_v, m3).wait()
    lanes = lax.iota(jnp.int32, 16)

    def _hist(k, cnt):
        sl = pl.ds(k * 16, 16)
        tid = tids_v[sl]
        hit = plsc.load_gather(table_v, [tid])
        age = ages_v[sl]
        keep = jnp.logical_and(
            jnp.logical_and(tid != 0, hit == 0.0), age <= HISTORY_LEN - 1)
        rows = start_h + k * 16 + lanes
        dropm = jnp.logical_not(keep)
        plsc.store_compressed(drop_v.at[pl.ds(cnt, 16)], rows, mask=dropm)
        return cnt + jnp.sum(dropm.astype(jnp.int32))
    cnt = lax.fori_loop(0, CH // 16, _hist, jnp.int32(0))

    pltpu.make_async_copy(ttids_hbm.at[pl.ds(start_t, CT)], ttids_v, m4).wait()

    def _det(k, cnt):
        sl = pl.ds(k * 16, 16)
        tid = ttids_v[sl]
        hit = plsc.load_gather(table_v, [tid])
        keep = jnp.logical_and(tid != 0, hit == 0.0)
        rows = start_t + k * 16 + lanes
        dropm = jnp.logical_not(keep)
        plsc.store_compressed(drop_v.at[pl.ds(cnt, 16)], rows + M, mask=dropm)
        return cnt + jnp.sum(dropm.astype(jnp.int32))
    cnt = lax.fori_loop(0, CT // 16, _det, cnt)

    @pl.loop(0, NSEG_H // NBUF)
    def _copy_h(o):
        for b in range(NBUF):
            g = o * NBUF + b
            pltpu.make_async_copy(h_src(0), bufs[b].at[pl.ds(0, SH), :],
                                  ins[b]).wait()
            pltpu.async_copy(bufs[b].at[pl.ds(0, SH), :], h_dst(g), outs[b])

            @pl.when(g + NBUF < NSEG_H)
            def _():
                pltpu.make_async_copy(bufs[b].at[pl.ds(0, SH), :], h_dst(0),
                                      outs[b]).wait()
                pltpu.async_copy(h_src(g + NBUF), bufs[b].at[pl.ds(0, SH), :],
                                 ins[b])

    for b in range(NBUF):
        pltpu.make_async_copy(bufs[b].at[pl.ds(0, SH), :], h_dst(0),
                              outs[b]).wait()

    def t_src(g):
        start = jnp.minimum(base_t + g * ST, N1 - ST)
        return tfeat_hbm.at[pl.ds(start, ST), :]

    def t_dst(g):
        start = jnp.minimum(base_t + g * ST, N1 - ST)
        return out_hbm.at[pl.ds(M + start, ST), :]

    for b in range(NBUF):
        pltpu.async_copy(t_src(b), bufs[b], ins[b])
    for g in range(NSEG_T):
        b = g % NBUF
        pltpu.make_async_copy(t_src(0), bufs[b], ins[b]).wait()
        pltpu.async_copy(bufs[b], t_dst(g), outs[b])
        if g + NBUF < NSEG_T:
            pltpu.make_async_copy(bufs[b], t_dst(0), outs[b]).wait()
            pltpu.async_copy(t_src(g + NBUF), bufs[b], ins[b])
    for g in range(max(NSEG_T - NBUF, 0), NSEG_T):
        pltpu.make_async_copy(bufs[g % NBUF], t_dst(0), outs[g % NBUF]).wait()

    pltpu.make_async_copy(zr_hbm, zrows_v, m5).wait()
    nfull = cnt // 16
    rem = cnt - nfull * 16

    def _fire(c, _):
        idx = drop_v[pl.ds(c * 16, 16)]
        pltpu.async_copy(zrows_v, out_hbm.at[idx], ssem)
        return 0
    lax.fori_loop(0, nfull, _fire, 0)

    @pl.when(rem > 0)
    def _():
        head = plsc.load_gather(drop_v, [jnp.full((16,), nfull * 16, jnp.int32)])
        tail = drop_v[pl.ds(nfull * 16, 16)]
        idx = jnp.where(lanes < rem, tail, head)
        pltpu.async_copy(zrows_v, out_hbm.at[idx], ssem)

    nchunks = nfull + jnp.where(rem > 0, 1, 0).astype(jnp.int32)

    def _drain(c, _):
        pltpu.make_async_copy(zrows_v, out_hbm.at[jnp.zeros((16,), jnp.int32)],
                              ssem).wait()
        return 0
    lax.fori_loop(0, nchunks, _drain, 0)


_sc_full = pl.kernel(
    _sc_body,
    out_type=jax.ShapeDtypeStruct((M + N1, D), jnp.float32),
    mesh=plsc.VectorSubcoreMesh(core_axis_name="c", subcore_axis_name="s"),
    compiler_params=pltpu.CompilerParams(needs_layout_passes=False),
    scratch_types=[
        pltpu.VMEM((TID_RANGE,), jnp.float32),
        pltpu.VMEM((NSET,), jnp.int32),
        pltpu.VMEM((CH,), jnp.int32),
        pltpu.VMEM((CH,), jnp.int32),
        pltpu.VMEM((CT,), jnp.int32),
        pltpu.VMEM((NDROP,), jnp.int32),
        pltpu.VMEM((16, D), jnp.float32),
    ]
    + [pltpu.VMEM((ST, D), jnp.float32)] * NBUF
    + [pltpu.SemaphoreType.DMA] * (2 * NBUF + 1 + 6),
)


def kernel(t1_feats, hist_feats, t1_tids, t2_tids, hist_tids, hist_ages):
    ztable = jnp.zeros((TID_RANGE,), jnp.float32)
    zrows = jnp.zeros((16, D), jnp.float32)
    return _sc_full(ztable, zrows, t2_tids[0], hist_tids[0], hist_ages,
                    t1_tids[0], hist_feats, t1_feats)


# speedup vs baseline: 2.2693x; 1.0091x over previous
import jax
import jax.numpy as jnp
from jax import lax
from jax.experimental import pallas as pl
from jax.experimental.pallas import tpu as pltpu
from jax.experimental.pallas import tpu_sc as plsc

M = 100000
N1 = 20000
D = 128
HISTORY_LEN = 30
TID_RANGE = 30000

NW = 32
CH = 3136
CT = 640
NSET = 512
NDROP = CH + CT + 16

SH = 112
NSEG_H = CH // SH
ST = 128
NSEG_T = CT // ST
NBUF = 4


def _sc_body(zr_hbm, t2_hbm, htids_hbm, hages_hbm, ttids_hbm,
             hfeat_hbm, tfeat_hbm, out_hbm,
             table_v, t2_v, tids_v, ages_v, ttids_v, drop_v, zrows_v,
             b0, b1, b2, b3, i0, i1, i2, i3, o0, o1, o2, o3, ssem,
             m1, m2, m3, m4, m5):
    wid = lax.axis_index("s") * 2 + lax.axis_index("c")
    bufs = [b0, b1, b2, b3]
    ins = [i0, i1, i2, i3]
    outs = [o0, o1, o2, o3]
    base_h = wid * CH
    start_h = jnp.minimum(base_h, M - CH)
    base_t = wid * CT
    start_t = jnp.minimum(base_t, N1 - CT)

    pltpu.async_copy(t2_hbm, t2_v, m1)
    pltpu.async_copy(htids_hbm.at[pl.ds(start_h, CH)], tids_v, m2)
    pltpu.async_copy(hages_hbm.at[pl.ds(start_h, CH)], ages_v, m3)
    pltpu.async_copy(ttids_hbm.at[pl.ds(start_t, CT)], ttids_v, m4)
    pltpu.async_copy(zr_hbm, zrows_v, m5)

    def h_src(g):
        start = jnp.minimum(base_h + g * SH, M - SH)
        return hfeat_hbm.at[pl.ds(start, SH), :]

    def h_dst(g):
        start = jnp.minimum(base_h + g * SH, M - SH)
        return out_hbm.at[pl.ds(start, SH), :]

    for b in range(NBUF):
        pltpu.async_copy(h_src(b), bufs[b].at[pl.ds(0, SH), :], ins[b])

    def _zt(i, _):
        table_v[pl.ds(i * 16, 16)] = jnp.zeros((16,), jnp.float32)
        return 0
    lax.fori_loop(0, TID_RANGE // 16, _zt, 0)

    pltpu.make_async_copy(t2_hbm, t2_v, m1).wait()

    def _scatter(j, _):
        idx = t2_v[pl.ds(j * 16, 16)]
        plsc.store_scatter(table_v, [idx], jnp.ones((16,), jnp.float32))
        return 0
    lax.fori_loop(0, NSET // 16, _scatter, 0)

    pltpu.make_async_copy(htids_hbm.at[pl.ds(start_h, CH)], tids_v, m2).wait()
    pltpu.make_async_copy(hages_hbm.at[pl.ds(start_h, CH)], ages_v, m3).wait()
    lanes = lax.iota(jnp.int32, 16)

    def _hist(k, cnt):
        sl = pl.ds(k * 16, 16)
        tid = tids_v[sl]
        hit = plsc.load_gather(table_v, [tid])
        age = ages_v[sl]
        keep = jnp.logical_and(
            jnp.logical_and(tid != 0, hit == 0.0), age <= HISTORY_LEN - 1)
        rows = start_h + k * 16 + lanes
        dropm = jnp.logical_not(keep)
        plsc.store_compressed(drop_v.at[pl.ds(cnt, 16)], rows, mask=dropm)
        return cnt + jnp.sum(dropm.astype(jnp.int32))
    cnt_h = lax.fori_loop(0, CH // 16, _hist, jnp.int32(0))

    pltpu.make_async_copy(ttids_hbm.at[pl.ds(start_t, CT)], ttids_v, m4).wait()

    def _det(k, cnt):
        sl = pl.ds(k * 16, 16)
        tid = ttids_v[sl]
        hit = plsc.load_gather(table_v, [tid])
        keep = jnp.logical_and(tid != 0, hit == 0.0)
        rows = start_t + k * 16 + lanes
        dropm = jnp.logical_not(keep)
        plsc.store_compressed(drop_v.at[pl.ds(cnt, 16)], rows + M, mask=dropm)
        return cnt + jnp.sum(dropm.astype(jnp.int32))
    cnt = lax.fori_loop(0, CT // 16, _det, cnt_h)

    @pl.loop(0, NSEG_H // NBUF)
    def _copy_h(o):
        for b in range(NBUF):
            g = o * NBUF + b
            pltpu.make_async_copy(h_src(0), bufs[b].at[pl.ds(0, SH), :],
                                  ins[b]).wait()
            pltpu.async_copy(bufs[b].at[pl.ds(0, SH), :], h_dst(g), outs[b])

            @pl.when(g + NBUF < NSEG_H)
            def _():
                pltpu.make_async_copy(bufs[b].at[pl.ds(0, SH), :], h_dst(0),
                                      outs[b]).wait()
                pltpu.async_copy(h_src(g + NBUF), bufs[b].at[pl.ds(0, SH), :],
                                 ins[b])

    for b in range(NBUF):
        pltpu.make_async_copy(bufs[b].at[pl.ds(0, SH), :], h_dst(0),
                              outs[b]).wait()

    pltpu.make_async_copy(zr_hbm, zrows_v, m5).wait()
    nfull_h = cnt_h // 16

    def _fire_h(c, _):
        idx = drop_v[pl.ds(c * 16, 16)]
        pltpu.async_copy(zrows_v, out_hbm.at[idx], ssem)
        return 0
    lax.fori_loop(0, nfull_h, _fire_h, 0)

    def t_src(g):
        start = jnp.minimum(base_t + g * ST, N1 - ST)
        return tfeat_hbm.at[pl.ds(start, ST), :]

    def t_dst(g):
        start = jnp.minimum(base_t + g * ST, N1 - ST)
        return out_hbm.at[pl.ds(M + start, ST), :]

    for b in range(NBUF):
        pltpu.async_copy(t_src(b), bufs[b], ins[b])
    for g in range(NSEG_T):
        b = g % NBUF
        pltpu.make_async_copy(t_src(0), bufs[b], ins[b]).wait()
        pltpu.async_copy(bufs[b], t_dst(g), outs[b])
        if g + NBUF < NSEG_T:
            pltpu.make_async_copy(bufs[b], t_dst(0), outs[b]).wait()
            pltpu.async_copy(t_src(g + NBUF), bufs[b], ins[b])
    for g in range(max(NSEG_T - NBUF, 0), NSEG_T):
        pltpu.make_async_copy(bufs[g % NBUF], t_dst(0), outs[g % NBUF]).wait()

    nfull = cnt // 16
    rem = cnt - nfull * 16

    def _fire(c, _):
        idx = drop_v[pl.ds(c * 16, 16)]
        pltpu.async_copy(zrows_v, out_hbm.at[idx], ssem)
        return 0
    lax.fori_loop(nfull_h, nfull, _fire, 0)

    @pl.when(rem > 0)
    def _():
        head = plsc.load_gather(drop_v, [jnp.full((16,), nfull * 16, jnp.int32)])
        tail = drop_v[pl.ds(nfull * 16, 16)]
        idx = jnp.where(lanes < rem, tail, head)
        pltpu.async_copy(zrows_v, out_hbm.at[idx], ssem)

    nchunks = nfull + jnp.where(rem > 0, 1, 0).astype(jnp.int32)

    def _drain(c, _):
        pltpu.make_async_copy(zrows_v, out_hbm.at[jnp.zeros((16,), jnp.int32)],
                              ssem).wait()
        return 0
    lax.fori_loop(0, nchunks, _drain, 0)


_sc_full = pl.kernel(
    _sc_body,
    out_type=jax.ShapeDtypeStruct((M + N1, D), jnp.float32),
    mesh=plsc.VectorSubcoreMesh(core_axis_name="c", subcore_axis_name="s"),
    compiler_params=pltpu.CompilerParams(needs_layout_passes=False),
    scratch_types=[
        pltpu.VMEM((TID_RANGE,), jnp.float32),
        pltpu.VMEM((NSET,), jnp.int32),
        pltpu.VMEM((CH,), jnp.int32),
        pltpu.VMEM((CH,), jnp.int32),
        pltpu.VMEM((CT,), jnp.int32),
        pltpu.VMEM((NDROP,), jnp.int32),
        pltpu.VMEM((16, D), jnp.float32),
    ]
    + [pltpu.VMEM((ST, D), jnp.float32)] * NBUF
    + [pltpu.SemaphoreType.DMA] * (2 * NBUF + 1 + 5),
)


def kernel(t1_feats, hist_feats, t1_tids, t2_tids, hist_tids, hist_ages):
    zrows = jnp.zeros((16, D), jnp.float32)
    return _sc_full(zrows, t2_tids[0], hist_tids[0], hist_ages,
                    t1_tids[0], hist_feats, t1_feats)
